# baseline (speedup 1.0000x reference)
import functools

import jax
import jax.numpy as jnp
from jax import lax
from jax.experimental import pallas as pl
from jax.experimental.pallas import tpu as pltpu
from jax.experimental.pallas import tpu_sc as plsc

N = 10000
D = 128
E = 320000
NC = 2
NS = 16
NW = NC * NS
EPW = E // NW
CH = 80
NCHUNK = EPW // CH
NBUF = 4
NGROUP = NCHUNK // NBUF
NREM = NCHUNK - NGROUP * NBUF
RPT = 632
RLAST0 = 15 * RPT
RLAST = N - RLAST0

_mesh = plsc.VectorSubcoreMesh(
    core_axis_name="c", subcore_axis_name="s", num_cores=NC, num_subcores=NS)


NPAD = 10240
SPT = NPAD // NS
CHD = 128
NCHD = E // CHD
CPW = NCHD // NW
NXTRA = NCHD - CPW * NW


@functools.partial(
    pl.kernel,
    mesh=_mesh,
    out_type=(
        jax.ShapeDtypeStruct((NC, NPAD), jnp.float32),
        jax.ShapeDtypeStruct((E,), jnp.int32),
        jax.ShapeDtypeStruct((E,), jnp.int32),
    ),
    scratch_types=[
        [pltpu.VMEM((2, CHD), jnp.int32)] * NBUF,
        pltpu.VMEM((CHD,), jnp.float32),
        pltpu.VMEM((SPT,), jnp.float32),
        [pltpu.SemaphoreType.DMA] * NBUF,
        [pltpu.SemaphoreType.DMA] * NBUF,
        [pltpu.SemaphoreType.DMA] * NBUF,
        pltpu.VMEM_SHARED((NPAD,), jnp.float32),
    ],
)
def _deg_kernel(ei_hbm, deg_hbm, src_hbm, dst_hbm, pbuf, ones_v, zbuf_v,
                isem, ssem, wsem, cnt_sh):
    c = lax.axis_index("c")
    s = lax.axis_index("s")
    wid = s * NC + c

    def zbody(i, carry):
        zbuf_v[pl.ds(i * 16, 16)] = jnp.zeros((16,), jnp.float32)
        return carry

    lax.fori_loop(0, SPT // 16, zbody, 0)

    def obody(i, carry):
        ones_v[pl.ds(i * 16, 16)] = jnp.ones((16,), jnp.float32)
        return carry

    lax.fori_loop(0, CHD // 16, obody, 0)

    pltpu.sync_copy(zbuf_v, cnt_sh.at[pl.ds(s * SPT, SPT)])
    plsc.subcore_barrier()

    first = wid * CPW + jnp.minimum(wid, NXTRA)

    def fetch_idx(q, b):
        pltpu.async_copy(ei_hbm.at[:, pl.ds(q * CHD, CHD)], pbuf[b], isem[b])

    def wait_idx(b):
        pltpu.make_async_copy(ei_hbm.at[:, pl.ds(0, CHD)], pbuf[b],
                              isem[b]).wait()

    def start_work(q, b):
        pltpu.async_copy(ones_v, cnt_sh.at[pbuf[b].at[1]], ssem[b], add=True)
        pltpu.async_copy(pbuf[b].at[0], src_hbm.at[pl.ds(q * CHD, CHD)],
                         wsem[b])
        pltpu.async_copy(pbuf[b].at[1], dst_hbm.at[pl.ds(q * CHD, CHD)],
                         wsem[b])

    def wait_work(b):
        pltpu.make_async_copy(ones_v, cnt_sh.at[pbuf[b].at[1]],
                              ssem[b]).wait()
        pltpu.make_async_copy(pbuf[b].at[0], src_hbm.at[pl.ds(0, CHD)],
                              wsem[b]).wait()
        pltpu.make_async_copy(pbuf[b].at[0], src_hbm.at[pl.ds(0, CHD)],
                              wsem[b]).wait()

    for b in range(NBUF):
        fetch_idx(first + b, b)

    def body(g, carry):
        for b in range(NBUF):
            q = first + g * NBUF + b
            wait_idx(b)
            start_work(q, b)
        for b in range(NBUF):
            nxt = g * NBUF + b + NBUF
            wait_work(b)

            @pl.when(nxt < CPW)
            def _():
                fetch_idx(first + nxt, b)

        return carry

    lax.fori_loop(0, CPW // NBUF, body, 0)
    for b in range(CPW - (CPW // NBUF) * NBUF):
        wait_idx(b)
        start_work(first + (CPW // NBUF) * NBUF + b, b)
        wait_work(b)

    @pl.when(wid < NXTRA)
    def _():
        q = first + CPW
        fetch_idx(q, 0)
        wait_idx(0)
        start_work(q, 0)
        wait_work(0)

    plsc.subcore_barrier()
    pltpu.sync_copy(cnt_sh.at[pl.ds(s * SPT, SPT)],
                    deg_hbm.at[c, pl.ds(s * SPT, SPT)])


@functools.partial(
    pl.kernel,
    mesh=_mesh,
    out_type=jax.ShapeDtypeStruct((NC, N, D), jnp.float32),
    scratch_types=[
        [pltpu.VMEM((CH,), jnp.int32)] * NBUF,
        [pltpu.VMEM((CH,), jnp.int32)] * NBUF,
        [pltpu.VMEM((CH, D), jnp.float32)] * NBUF,
        [pltpu.SemaphoreType.DMA] * NBUF,
        [pltpu.SemaphoreType.DMA] * NBUF,
        [pltpu.SemaphoreType.DMA] * NBUF,
        pltpu.VMEM_SHARED((N, D), jnp.float32),
    ],
)
def _agg_kernel(hs_hbm, src_hbm, dst_hbm, out_hbm, sbuf, dbuf, rows,
                isem, gsem, ssem, agg_sh):
    c = lax.axis_index("c")
    s = lax.axis_index("s")
    wid = s * NC + c
    r0 = s * RPT

    @pl.when(s < NS - 1)
    def _():
        pltpu.sync_copy(hs_hbm.at[pl.ds(r0, RPT)], agg_sh.at[pl.ds(r0, RPT)])

    @pl.when(s == NS - 1)
    def _():
        pltpu.sync_copy(hs_hbm.at[pl.ds(RLAST0, RLAST)],
                        agg_sh.at[pl.ds(RLAST0, RLAST)])

    plsc.subcore_barrier()

    base = wid * EPW

    def fetch_idx(i, b):
        off = base + i * CH
        pltpu.async_copy(src_hbm.at[pl.ds(off, CH)], sbuf[b], isem[b])
        pltpu.async_copy(dst_hbm.at[pl.ds(off, CH)], dbuf[b], isem[b])

    def wait_idx(b):
        pltpu.make_async_copy(src_hbm.at[pl.ds(0, CH)], sbuf[b],
                              isem[b]).wait()
        pltpu.make_async_copy(dst_hbm.at[pl.ds(0, CH)], dbuf[b],
                              isem[b]).wait()

    def start_gather(b):
        pltpu.async_copy(hs_hbm.at[sbuf[b]], rows[b], gsem[b])

    def wait_gather(b):
        pltpu.make_async_copy(hs_hbm.at[pl.ds(0, CH)], rows[b],
                              gsem[b]).wait()

    def start_scatter(b):
        pltpu.async_copy(rows[b], agg_sh.at[dbuf[b]], ssem[b], add=True)

    def wait_scatter(b):
        pltpu.make_async_copy(rows[b], agg_sh.at[dbuf[b]], ssem[b]).wait()

    for b in range(NBUF):
        fetch_idx(b, b)

    def body(g, carry):
        for b in range(NBUF):
            wait_idx(b)
            start_gather(b)
        for b in range(NBUF):
            wait_gather(b)
            start_scatter(b)
        for b in range(NBUF):
            nxt = g * NBUF + b + NBUF
            wait_scatter(b)

            @pl.when(nxt < NCHUNK)
            def _():
                fetch_idx(nxt, b)

        return carry

    lax.fori_loop(0, NGROUP, body, 0)
    for b in range(NREM):
        wait_idx(b)
        start_gather(b)
    for b in range(NREM):
        wait_gather(b)
        start_scatter(b)
    for b in range(NREM):
        wait_scatter(b)
    plsc.subcore_barrier()

    @pl.when(s < NS - 1)
    def _():
        pltpu.sync_copy(agg_sh.at[pl.ds(r0, RPT)],
                        out_hbm.at[c, pl.ds(r0, RPT)])

    @pl.when(s == NS - 1)
    def _():
        pltpu.sync_copy(agg_sh.at[pl.ds(RLAST0, RLAST)],
                        out_hbm.at[c, pl.ds(RLAST0, RLAST)])


def _dinv_from(degs_block):
    deg = jnp.sum(degs_block, axis=0)[:N] + 1.0
    return lax.rsqrt(deg)


def _tc0_body(x_ref, w1_ref, mm_ref):
    mm_ref[...] = jnp.dot(x_ref[...], w1_ref[...],
                          preferred_element_type=jnp.float32)


def _tc1_body(degs_ref, mm_ref, hs_ref):
    dinv = _dinv_from(degs_ref[...])
    hs_ref[...] = mm_ref[...] * dinv[:, None]


def _tc2_body(degs_ref, agg_ref, hs1_ref, b1_ref, w2_ref, hs2_ref):
    dinv = _dinv_from(degs_ref[...])
    tot = agg_ref[0] + agg_ref[1] - hs1_ref[...]
    h1 = jnp.maximum(tot * dinv[:, None] + b1_ref[...][None, :], 0.0)
    hs2_ref[...] = jnp.dot(h1, w2_ref[...],
                           preferred_element_type=jnp.float32) * dinv[:, None]


def _tc3_body(degs_ref, agg_ref, hs2_ref, b2_ref, wh1_ref, bh1_ref, wh2_ref,
              bh2_ref, out1_ref, out2_ref, h_ref):
    dinv = _dinv_from(degs_ref[...])
    tot = agg_ref[0] + agg_ref[1] - hs2_ref[...]
    h = tot * dinv[:, None] + b2_ref[...][None, :]
    h_ref[...] = h
    out1_ref[...] = jnp.dot(h, wh1_ref[...],
                            preferred_element_type=jnp.float32) + bh1_ref[...][None, :]
    out2_ref[...] = jnp.dot(h, wh2_ref[...],
                            preferred_element_type=jnp.float32) + bh2_ref[...][None, :]


def kernel(x, edge_index, W1, b1, W2, b2, Wh1, bh1, Wh2, bh2):
    ei32 = edge_index.astype(jnp.int32)

    degs, src, dst = _deg_kernel(ei32)

    mm1 = pl.pallas_call(
        _tc0_body,
        out_shape=jax.ShapeDtypeStruct((N, D), jnp.float32),
    )(x, W1)

    hs1 = pl.pallas_call(
        _tc1_body,
        out_shape=jax.ShapeDtypeStruct((N, D), jnp.float32),
    )(degs, mm1)

    agg1 = _agg_kernel(hs1, src, dst)

    hs2 = pl.pallas_call(
        _tc2_body,
        out_shape=jax.ShapeDtypeStruct((N, D), jnp.float32),
    )(degs, agg1, hs1, b1, W2)

    agg2 = _agg_kernel(hs2, src, dst)

    out1, out2, h = pl.pallas_call(
        _tc3_body,
        out_shape=(
            jax.ShapeDtypeStruct((N, Wh1.shape[1]), jnp.float32),
            jax.ShapeDtypeStruct((N, Wh2.shape[1]), jnp.float32),
            jax.ShapeDtypeStruct((N, D), jnp.float32),
        ),
    )(degs, agg2, hs2, b2, Wh1, bh1, Wh2, bh2)

    return out1, out2, h


# speedup vs baseline: 1.0063x; 1.0063x over previous
import functools

import jax
import jax.numpy as jnp
from jax import lax
from jax.experimental import pallas as pl
from jax.experimental.pallas import tpu as pltpu
from jax.experimental.pallas import tpu_sc as plsc

N = 10000
D = 128
E = 320000
NC = 2
NS = 16
NW = NC * NS
EPW = E // NW
CH = 80
NCHUNK = EPW // CH
NBUF = 4
NGROUP = NCHUNK // NBUF
NREM = NCHUNK - NGROUP * NBUF
RPT = 632
RLAST0 = 15 * RPT
RLAST = N - RLAST0

_mesh = plsc.VectorSubcoreMesh(
    core_axis_name="c", subcore_axis_name="s", num_cores=NC, num_subcores=NS)


NPAD = 10240
SPT = NPAD // NS
CHD = 128
NCHD = E // CHD
CPW = NCHD // NW
NXTRA = NCHD - CPW * NW


@functools.partial(
    pl.kernel,
    mesh=_mesh,
    out_type=(
        jax.ShapeDtypeStruct((NC, NPAD), jnp.float32),
        jax.ShapeDtypeStruct((E,), jnp.int32),
        jax.ShapeDtypeStruct((E,), jnp.int32),
    ),
    scratch_types=[
        [pltpu.VMEM((2, CHD), jnp.int32)] * NBUF,
        pltpu.VMEM((CHD,), jnp.float32),
        pltpu.VMEM((SPT,), jnp.float32),
        [pltpu.SemaphoreType.DMA] * NBUF,
        [pltpu.SemaphoreType.DMA] * NBUF,
        [pltpu.SemaphoreType.DMA] * NBUF,
        pltpu.VMEM_SHARED((NPAD,), jnp.float32),
    ],
)
def _deg_kernel(ei_hbm, deg_hbm, src_hbm, dst_hbm, pbuf, ones_v, zbuf_v,
                isem, ssem, wsem, cnt_sh):
    c = lax.axis_index("c")
    s = lax.axis_index("s")
    wid = s * NC + c

    def zbody(i, carry):
        zbuf_v[pl.ds(i * 16, 16)] = jnp.zeros((16,), jnp.float32)
        return carry

    lax.fori_loop(0, SPT // 16, zbody, 0)

    def obody(i, carry):
        ones_v[pl.ds(i * 16, 16)] = jnp.ones((16,), jnp.float32)
        return carry

    lax.fori_loop(0, CHD // 16, obody, 0)

    pltpu.sync_copy(zbuf_v, cnt_sh.at[pl.ds(s * SPT, SPT)])
    plsc.subcore_barrier()

    first = wid * CPW + jnp.minimum(wid, NXTRA)

    def fetch_idx(q, b):
        pltpu.async_copy(ei_hbm.at[:, pl.ds(q * CHD, CHD)], pbuf[b], isem[b])

    def wait_idx(b):
        pltpu.make_async_copy(ei_hbm.at[:, pl.ds(0, CHD)], pbuf[b],
                              isem[b]).wait()

    def start_work(q, b):
        pltpu.async_copy(ones_v, cnt_sh.at[pbuf[b].at[1]], ssem[b], add=True)
        pltpu.async_copy(pbuf[b].at[0], src_hbm.at[pl.ds(q * CHD, CHD)],
                         wsem[b])
        pltpu.async_copy(pbuf[b].at[1], dst_hbm.at[pl.ds(q * CHD, CHD)],
                         wsem[b])

    def wait_work(b):
        pltpu.make_async_copy(ones_v, cnt_sh.at[pbuf[b].at[1]],
                              ssem[b]).wait()
        pltpu.make_async_copy(pbuf[b].at[0], src_hbm.at[pl.ds(0, CHD)],
                              wsem[b]).wait()
        pltpu.make_async_copy(pbuf[b].at[0], src_hbm.at[pl.ds(0, CHD)],
                              wsem[b]).wait()

    for b in range(NBUF):
        fetch_idx(first + b, b)

    def body(g, carry):
        for b in range(NBUF):
            q = first + g * NBUF + b
            wait_idx(b)
            start_work(q, b)
        for b in range(NBUF):
            nxt = g * NBUF + b + NBUF
            wait_work(b)

            @pl.when(nxt < CPW)
            def _():
                fetch_idx(first + nxt, b)

        return carry

    lax.fori_loop(0, CPW // NBUF, body, 0)
    for b in range(CPW - (CPW // NBUF) * NBUF):
        wait_idx(b)
        start_work(first + (CPW // NBUF) * NBUF + b, b)
        wait_work(b)

    @pl.when(wid < NXTRA)
    def _():
        q = first + CPW
        fetch_idx(q, 0)
        wait_idx(0)
        start_work(q, 0)
        wait_work(0)

    plsc.subcore_barrier()
    pltpu.sync_copy(cnt_sh.at[pl.ds(s * SPT, SPT)],
                    deg_hbm.at[c, pl.ds(s * SPT, SPT)])


@functools.partial(
    pl.kernel,
    mesh=_mesh,
    out_type=jax.ShapeDtypeStruct((NC, N, D), jnp.float32),
    scratch_types=[
        [pltpu.VMEM((CH,), jnp.int32)] * NBUF,
        [pltpu.VMEM((CH,), jnp.int32)] * NBUF,
        [pltpu.VMEM((CH, D), jnp.float32)] * NBUF,
        [pltpu.SemaphoreType.DMA] * NBUF,
        [pltpu.SemaphoreType.DMA] * NBUF,
        [pltpu.SemaphoreType.DMA] * NBUF,
        pltpu.VMEM_SHARED((N, D), jnp.float32),
    ],
)
def _agg_kernel(hs_hbm, src_hbm, dst_hbm, out_hbm, sbuf, dbuf, rows,
                isem, gsem, ssem, agg_sh):
    c = lax.axis_index("c")
    s = lax.axis_index("s")
    wid = s * NC + c
    r0 = s * RPT
    base = wid * EPW

    def fetch_idx(i, b):
        off = base + i * CH
        pltpu.async_copy(src_hbm.at[pl.ds(off, CH)], sbuf[b], isem[b])
        pltpu.async_copy(dst_hbm.at[pl.ds(off, CH)], dbuf[b], isem[b])

    def wait_idx(b):
        pltpu.make_async_copy(src_hbm.at[pl.ds(0, CH)], sbuf[b],
                              isem[b]).wait()
        pltpu.make_async_copy(dst_hbm.at[pl.ds(0, CH)], dbuf[b],
                              isem[b]).wait()

    def start_gather(b):
        pltpu.async_copy(hs_hbm.at[sbuf[b]], rows[b], gsem[b])

    def wait_gather(b):
        pltpu.make_async_copy(hs_hbm.at[pl.ds(0, CH)], rows[b],
                              gsem[b]).wait()

    def start_scatter(b):
        pltpu.async_copy(rows[b], agg_sh.at[dbuf[b]], ssem[b], add=True)

    def wait_scatter(b):
        pltpu.make_async_copy(rows[b], agg_sh.at[dbuf[b]], ssem[b]).wait()

    for b in range(NBUF):
        fetch_idx(b, b)

    @pl.when(s < NS - 1)
    def _():
        pltpu.sync_copy(hs_hbm.at[pl.ds(r0, RPT)], agg_sh.at[pl.ds(r0, RPT)])

    @pl.when(s == NS - 1)
    def _():
        pltpu.sync_copy(hs_hbm.at[pl.ds(RLAST0, RLAST)],
                        agg_sh.at[pl.ds(RLAST0, RLAST)])

    plsc.subcore_barrier()

    def body(g, carry):
        for b in range(NBUF):
            wait_idx(b)
            start_gather(b)
        for b in range(NBUF):
            wait_gather(b)
            start_scatter(b)
        for b in range(NBUF):
            nxt = g * NBUF + b + NBUF
            wait_scatter(b)

            @pl.when(nxt < NCHUNK)
            def _():
                fetch_idx(nxt, b)

        return carry

    lax.fori_loop(0, NGROUP, body, 0)
    for b in range(NREM):
        wait_idx(b)
        start_gather(b)
    for b in range(NREM):
        wait_gather(b)
        start_scatter(b)
    for b in range(NREM):
        wait_scatter(b)
    plsc.subcore_barrier()

    @pl.when(s < NS - 1)
    def _():
        pltpu.sync_copy(agg_sh.at[pl.ds(r0, RPT)],
                        out_hbm.at[c, pl.ds(r0, RPT)])

    @pl.when(s == NS - 1)
    def _():
        pltpu.sync_copy(agg_sh.at[pl.ds(RLAST0, RLAST)],
                        out_hbm.at[c, pl.ds(RLAST0, RLAST)])


def _dinv_from(degs_block):
    deg = jnp.sum(degs_block, axis=0)[:N] + 1.0
    return lax.rsqrt(deg)


def _tc1_body(degs_ref, x_ref, w1_ref, hs_ref):
    dinv = _dinv_from(degs_ref[...])
    hs_ref[...] = jnp.dot(x_ref[...], w1_ref[...],
                          preferred_element_type=jnp.float32) * dinv[:, None]


def _tc2_body(degs_ref, agg_ref, hs1_ref, b1_ref, w2_ref, hs2_ref):
    dinv = _dinv_from(degs_ref[...])
    tot = agg_ref[0] + agg_ref[1] - hs1_ref[...]
    h1 = jnp.maximum(tot * dinv[:, None] + b1_ref[...][None, :], 0.0)
    hs2_ref[...] = jnp.dot(h1, w2_ref[...],
                           preferred_element_type=jnp.float32) * dinv[:, None]


def _tc3_body(degs_ref, agg_ref, hs2_ref, b2_ref, wh1_ref, bh1_ref, wh2_ref,
              bh2_ref, out1_ref, out2_ref, h_ref):
    dinv = _dinv_from(degs_ref[...])
    tot = agg_ref[0] + agg_ref[1] - hs2_ref[...]
    h = tot * dinv[:, None] + b2_ref[...][None, :]
    h_ref[...] = h
    out1_ref[...] = jnp.dot(h, wh1_ref[...],
                            preferred_element_type=jnp.float32) + bh1_ref[...][None, :]
    out2_ref[...] = jnp.dot(h, wh2_ref[...],
                            preferred_element_type=jnp.float32) + bh2_ref[...][None, :]


def kernel(x, edge_index, W1, b1, W2, b2, Wh1, bh1, Wh2, bh2):
    ei32 = edge_index.astype(jnp.int32)

    degs, src, dst = _deg_kernel(ei32)

    hs1 = pl.pallas_call(
        _tc1_body,
        out_shape=jax.ShapeDtypeStruct((N, D), jnp.float32),
    )(degs, x, W1)

    agg1 = _agg_kernel(hs1, src, dst)

    hs2 = pl.pallas_call(
        _tc2_body,
        out_shape=jax.ShapeDtypeStruct((N, D), jnp.float32),
    )(degs, agg1, hs1, b1, W2)

    agg2 = _agg_kernel(hs2, src, dst)

    out1, out2, h = pl.pallas_call(
        _tc3_body,
        out_shape=(
            jax.ShapeDtypeStruct((N, Wh1.shape[1]), jnp.float32),
            jax.ShapeDtypeStruct((N, Wh2.shape[1]), jnp.float32),
            jax.ShapeDtypeStruct((N, D), jnp.float32),
        ),
    )(degs, agg2, hs2, b2, Wh1, bh1, Wh2, bh2)

    return out1, out2, h


# speedup vs baseline: 1.0174x; 1.0111x over previous
import functools

import jax
import jax.numpy as jnp
from jax import lax
from jax.experimental import pallas as pl
from jax.experimental.pallas import tpu as pltpu
from jax.experimental.pallas import tpu_sc as plsc

N = 10000
D = 128
E = 320000
NC = 2
NS = 16
NW = NC * NS
EPW = E // NW
CH = 80
NCHUNK = EPW // CH
NBUF = 4
NGROUP = NCHUNK // NBUF
NREM = NCHUNK - NGROUP * NBUF
RPT = 632
RLAST0 = 15 * RPT
RLAST = N - RLAST0

_mesh = plsc.VectorSubcoreMesh(
    core_axis_name="c", subcore_axis_name="s", num_cores=NC, num_subcores=NS)


NPAD = 10240
SPT = NPAD // NS
CHD = 256
NCHD = E // CHD
CPW = NCHD // NW
NXTRA = NCHD - CPW * NW


@functools.partial(
    pl.kernel,
    mesh=_mesh,
    out_type=(
        jax.ShapeDtypeStruct((NC, NPAD), jnp.float32),
        jax.ShapeDtypeStruct((E,), jnp.int32),
        jax.ShapeDtypeStruct((E,), jnp.int32),
    ),
    scratch_types=[
        [pltpu.VMEM((2, CHD), jnp.int32)] * NBUF,
        pltpu.VMEM((CHD,), jnp.float32),
        pltpu.VMEM((SPT,), jnp.float32),
        [pltpu.SemaphoreType.DMA] * NBUF,
        [pltpu.SemaphoreType.DMA] * NBUF,
        [pltpu.SemaphoreType.DMA] * NBUF,
        pltpu.VMEM_SHARED((NPAD,), jnp.float32),
    ],
)
def _deg_kernel(ei_hbm, deg_hbm, src_hbm, dst_hbm, pbuf, ones_v, zbuf_v,
                isem, ssem, wsem, cnt_sh):
    c = lax.axis_index("c")
    s = lax.axis_index("s")
    wid = s * NC + c

    def zbody(i, carry):
        zbuf_v[pl.ds(i * 16, 16)] = jnp.zeros((16,), jnp.float32)
        return carry

    lax.fori_loop(0, SPT // 16, zbody, 0)

    def obody(i, carry):
        ones_v[pl.ds(i * 16, 16)] = jnp.ones((16,), jnp.float32)
        return carry

    lax.fori_loop(0, CHD // 16, obody, 0)

    pltpu.sync_copy(zbuf_v, cnt_sh.at[pl.ds(s * SPT, SPT)])
    plsc.subcore_barrier()

    first = wid * CPW + jnp.minimum(wid, NXTRA)

    def fetch_idx(q, b):
        pltpu.async_copy(ei_hbm.at[:, pl.ds(q * CHD, CHD)], pbuf[b], isem[b])

    def wait_idx(b):
        pltpu.make_async_copy(ei_hbm.at[:, pl.ds(0, CHD)], pbuf[b],
                              isem[b]).wait()

    def start_work(q, b):
        pltpu.async_copy(ones_v.at[pl.ds(0, 128)],
                         cnt_sh.at[pbuf[b].at[1, pl.ds(0, 128)]],
                         ssem[b], add=True)
        pltpu.async_copy(ones_v.at[pl.ds(0, 128)],
                         cnt_sh.at[pbuf[b].at[1, pl.ds(128, 128)]],
                         ssem[b], add=True)
        pltpu.async_copy(pbuf[b].at[0], src_hbm.at[pl.ds(q * CHD, CHD)],
                         wsem[b])
        pltpu.async_copy(pbuf[b].at[1], dst_hbm.at[pl.ds(q * CHD, CHD)],
                         wsem[b])

    def wait_work(b):
        pltpu.make_async_copy(ones_v.at[pl.ds(0, 128)],
                              cnt_sh.at[pbuf[b].at[1, pl.ds(0, 128)]],
                              ssem[b]).wait()
        pltpu.make_async_copy(ones_v.at[pl.ds(0, 128)],
                              cnt_sh.at[pbuf[b].at[1, pl.ds(0, 128)]],
                              ssem[b]).wait()
        pltpu.make_async_copy(pbuf[b].at[0], src_hbm.at[pl.ds(0, CHD)],
                              wsem[b]).wait()
        pltpu.make_async_copy(pbuf[b].at[0], src_hbm.at[pl.ds(0, CHD)],
                              wsem[b]).wait()

    for b in range(NBUF):
        fetch_idx(first + b, b)

    def body(g, carry):
        for b in range(NBUF):
            q = first + g * NBUF + b
            wait_idx(b)
            start_work(q, b)
        for b in range(NBUF):
            nxt = g * NBUF + b + NBUF
            wait_work(b)

            @pl.when(nxt < CPW)
            def _():
                fetch_idx(first + nxt, b)

        return carry

    lax.fori_loop(0, CPW // NBUF, body, 0)
    for b in range(CPW - (CPW // NBUF) * NBUF):
        wait_idx(b)
        start_work(first + (CPW // NBUF) * NBUF + b, b)
        wait_work(b)

    @pl.when(wid < NXTRA)
    def _():
        q = first + CPW
        fetch_idx(q, 0)
        wait_idx(0)
        start_work(q, 0)
        wait_work(0)

    plsc.subcore_barrier()
    pltpu.sync_copy(cnt_sh.at[pl.ds(s * SPT, SPT)],
                    deg_hbm.at[c, pl.ds(s * SPT, SPT)])


@functools.partial(
    pl.kernel,
    mesh=_mesh,
    out_type=jax.ShapeDtypeStruct((NC, N, D), jnp.float32),
    scratch_types=[
        [pltpu.VMEM((CH,), jnp.int32)] * NBUF,
        [pltpu.VMEM((CH,), jnp.int32)] * NBUF,
        [pltpu.VMEM((CH, D), jnp.float32)] * NBUF,
        [pltpu.SemaphoreType.DMA] * NBUF,
        [pltpu.SemaphoreType.DMA] * NBUF,
        [pltpu.SemaphoreType.DMA] * NBUF,
        pltpu.VMEM_SHARED((N, D), jnp.float32),
    ],
)
def _agg_kernel(hs_hbm, src_hbm, dst_hbm, out_hbm, sbuf, dbuf, rows,
                isem, gsem, ssem, agg_sh):
    c = lax.axis_index("c")
    s = lax.axis_index("s")
    wid = s * NC + c
    r0 = s * RPT
    base = wid * EPW

    def fetch_idx(i, b):
        off = base + i * CH
        pltpu.async_copy(src_hbm.at[pl.ds(off, CH)], sbuf[b], isem[b])
        pltpu.async_copy(dst_hbm.at[pl.ds(off, CH)], dbuf[b], isem[b])

    def wait_idx(b):
        pltpu.make_async_copy(src_hbm.at[pl.ds(0, CH)], sbuf[b],
                              isem[b]).wait()
        pltpu.make_async_copy(dst_hbm.at[pl.ds(0, CH)], dbuf[b],
                              isem[b]).wait()

    def start_gather(b):
        pltpu.async_copy(hs_hbm.at[sbuf[b]], rows[b], gsem[b])

    def wait_gather(b):
        pltpu.make_async_copy(hs_hbm.at[pl.ds(0, CH)], rows[b],
                              gsem[b]).wait()

    def start_scatter(b):
        pltpu.async_copy(rows[b], agg_sh.at[dbuf[b]], ssem[b], add=True)

    def wait_scatter(b):
        pltpu.make_async_copy(rows[b], agg_sh.at[dbuf[b]], ssem[b]).wait()

    for b in range(NBUF):
        fetch_idx(b, b)

    @pl.when(s < NS - 1)
    def _():
        pltpu.sync_copy(hs_hbm.at[pl.ds(r0, RPT)], agg_sh.at[pl.ds(r0, RPT)])

    @pl.when(s == NS - 1)
    def _():
        pltpu.sync_copy(hs_hbm.at[pl.ds(RLAST0, RLAST)],
                        agg_sh.at[pl.ds(RLAST0, RLAST)])

    plsc.subcore_barrier()

    def body(g, carry):
        for b in range(NBUF):
            wait_idx(b)
            start_gather(b)
        for b in range(NBUF):
            wait_gather(b)
            start_scatter(b)
        for b in range(NBUF):
            nxt = g * NBUF + b + NBUF
            wait_scatter(b)

            @pl.when(nxt < NCHUNK)
            def _():
                fetch_idx(nxt, b)

        return carry

    lax.fori_loop(0, NGROUP, body, 0)
    for b in range(NREM):
        wait_idx(b)
        start_gather(b)
    for b in range(NREM):
        wait_gather(b)
        start_scatter(b)
    for b in range(NREM):
        wait_scatter(b)
    plsc.subcore_barrier()

    @pl.when(s < NS - 1)
    def _():
        pltpu.sync_copy(agg_sh.at[pl.ds(r0, RPT)],
                        out_hbm.at[c, pl.ds(r0, RPT)])

    @pl.when(s == NS - 1)
    def _():
        pltpu.sync_copy(agg_sh.at[pl.ds(RLAST0, RLAST)],
                        out_hbm.at[c, pl.ds(RLAST0, RLAST)])


def _dinv_from(degs_block):
    deg = jnp.sum(degs_block, axis=0)[:N] + 1.0
    return lax.rsqrt(deg)


def _tc1_body(degs_ref, x_ref, w1_ref, hs_ref):
    dinv = _dinv_from(degs_ref[...])
    hs_ref[...] = jnp.dot(x_ref[...], w1_ref[...],
                          preferred_element_type=jnp.float32) * dinv[:, None]


def _tc2_body(degs_ref, agg_ref, hs1_ref, b1_ref, w2_ref, hs2_ref):
    dinv = _dinv_from(degs_ref[...])
    tot = agg_ref[0] + agg_ref[1] - hs1_ref[...]
    h1 = jnp.maximum(tot * dinv[:, None] + b1_ref[...][None, :], 0.0)
    hs2_ref[...] = jnp.dot(h1, w2_ref[...],
                           preferred_element_type=jnp.float32) * dinv[:, None]


def _tc3_body(degs_ref, agg_ref, hs2_ref, b2_ref, wh1_ref, bh1_ref, wh2_ref,
              bh2_ref, out1_ref, out2_ref, h_ref):
    dinv = _dinv_from(degs_ref[...])
    tot = agg_ref[0] + agg_ref[1] - hs2_ref[...]
    h = tot * dinv[:, None] + b2_ref[...][None, :]
    h_ref[...] = h
    out1_ref[...] = jnp.dot(h, wh1_ref[...],
                            preferred_element_type=jnp.float32) + bh1_ref[...][None, :]
    out2_ref[...] = jnp.dot(h, wh2_ref[...],
                            preferred_element_type=jnp.float32) + bh2_ref[...][None, :]


def kernel(x, edge_index, W1, b1, W2, b2, Wh1, bh1, Wh2, bh2):
    ei32 = edge_index.astype(jnp.int32)

    degs, src, dst = _deg_kernel(ei32)

    hs1 = pl.pallas_call(
        _tc1_body,
        out_shape=jax.ShapeDtypeStruct((N, D), jnp.float32),
    )(degs, x, W1)

    agg1 = _agg_kernel(hs1, src, dst)

    hs2 = pl.pallas_call(
        _tc2_body,
        out_shape=jax.ShapeDtypeStruct((N, D), jnp.float32),
    )(degs, agg1, hs1, b1, W2)

    agg2 = _agg_kernel(hs2, src, dst)

    out1, out2, h = pl.pallas_call(
        _tc3_body,
        out_shape=(
            jax.ShapeDtypeStruct((N, Wh1.shape[1]), jnp.float32),
            jax.ShapeDtypeStruct((N, Wh2.shape[1]), jnp.float32),
            jax.ShapeDtypeStruct((N, D), jnp.float32),
        ),
    )(degs, agg2, hs2, b2, Wh1, bh1, Wh2, bh2)

    return out1, out2, h


# speedup vs baseline: 1.0284x; 1.0108x over previous
import functools

import jax
import jax.numpy as jnp
from jax import lax
from jax.experimental import pallas as pl
from jax.experimental.pallas import tpu as pltpu
from jax.experimental.pallas import tpu_sc as plsc

N = 10000
D = 128
E = 320000
NC = 2
NS = 16
NW = NC * NS
EPW = E // NW
CH = 80
NCHUNK = EPW // CH
NBUF = 4
NGROUP = NCHUNK // NBUF
NREM = NCHUNK - NGROUP * NBUF
RPT = 632
RLAST0 = 15 * RPT
RLAST = N - RLAST0

_mesh = plsc.VectorSubcoreMesh(
    core_axis_name="c", subcore_axis_name="s", num_cores=NC, num_subcores=NS)


NPAD = 10240
SPT = NPAD // NS
CHD = 512
NCHD = E // CHD
CPW = NCHD // NW
NXTRA = NCHD - CPW * NW


@functools.partial(
    pl.kernel,
    mesh=_mesh,
    out_type=(
        jax.ShapeDtypeStruct((NC, NPAD), jnp.float32),
        jax.ShapeDtypeStruct((E,), jnp.int32),
        jax.ShapeDtypeStruct((E,), jnp.int32),
    ),
    scratch_types=[
        [pltpu.VMEM((2, CHD), jnp.int32)] * NBUF,
        pltpu.VMEM((CHD,), jnp.float32),
        pltpu.VMEM((SPT,), jnp.float32),
        [pltpu.SemaphoreType.DMA] * NBUF,
        [pltpu.SemaphoreType.DMA] * NBUF,
        [pltpu.SemaphoreType.DMA] * NBUF,
        pltpu.VMEM_SHARED((NPAD,), jnp.float32),
    ],
)
def _deg_kernel(ei_hbm, deg_hbm, src_hbm, dst_hbm, pbuf, ones_v, zbuf_v,
                isem, ssem, wsem, cnt_sh):
    c = lax.axis_index("c")
    s = lax.axis_index("s")
    wid = s * NC + c

    def zbody(i, carry):
        zbuf_v[pl.ds(i * 16, 16)] = jnp.zeros((16,), jnp.float32)
        return carry

    lax.fori_loop(0, SPT // 16, zbody, 0)

    def obody(i, carry):
        ones_v[pl.ds(i * 16, 16)] = jnp.ones((16,), jnp.float32)
        return carry

    lax.fori_loop(0, CHD // 16, obody, 0)

    pltpu.sync_copy(zbuf_v, cnt_sh.at[pl.ds(s * SPT, SPT)])
    plsc.subcore_barrier()

    first = wid * CPW + jnp.minimum(wid, NXTRA)

    def fetch_idx(q, b):
        pltpu.async_copy(ei_hbm.at[:, pl.ds(q * CHD, CHD)], pbuf[b], isem[b])

    def wait_idx(b):
        pltpu.make_async_copy(ei_hbm.at[:, pl.ds(0, CHD)], pbuf[b],
                              isem[b]).wait()

    def start_work(q, b):
        for h in range(CHD // 128):
            pltpu.async_copy(ones_v.at[pl.ds(0, 128)],
                             cnt_sh.at[pbuf[b].at[1, pl.ds(h * 128, 128)]],
                             ssem[b], add=True)
        pltpu.async_copy(pbuf[b].at[0], src_hbm.at[pl.ds(q * CHD, CHD)],
                         wsem[b])
        pltpu.async_copy(pbuf[b].at[1], dst_hbm.at[pl.ds(q * CHD, CHD)],
                         wsem[b])

    def wait_work(b):
        for _h in range(CHD // 128):
            pltpu.make_async_copy(ones_v.at[pl.ds(0, 128)],
                                  cnt_sh.at[pbuf[b].at[1, pl.ds(0, 128)]],
                                  ssem[b]).wait()
        pltpu.make_async_copy(pbuf[b].at[0], src_hbm.at[pl.ds(0, CHD)],
                              wsem[b]).wait()
        pltpu.make_async_copy(pbuf[b].at[0], src_hbm.at[pl.ds(0, CHD)],
                              wsem[b]).wait()

    for b in range(NBUF):
        fetch_idx(first + b, b)

    def body(g, carry):
        for b in range(NBUF):
            q = first + g * NBUF + b
            wait_idx(b)
            start_work(q, b)
        for b in range(NBUF):
            nxt = g * NBUF + b + NBUF
            wait_work(b)

            @pl.when(nxt < CPW)
            def _():
                fetch_idx(first + nxt, b)

        return carry

    lax.fori_loop(0, CPW // NBUF, body, 0)
    for b in range(CPW - (CPW // NBUF) * NBUF):
        wait_idx(b)
        start_work(first + (CPW // NBUF) * NBUF + b, b)
        wait_work(b)

    @pl.when(wid < NXTRA)
    def _():
        q = first + CPW
        fetch_idx(q, 0)
        wait_idx(0)
        start_work(q, 0)
        wait_work(0)

    plsc.subcore_barrier()
    pltpu.sync_copy(cnt_sh.at[pl.ds(s * SPT, SPT)],
                    deg_hbm.at[c, pl.ds(s * SPT, SPT)])


@functools.partial(
    pl.kernel,
    mesh=_mesh,
    out_type=jax.ShapeDtypeStruct((NC, N, D), jnp.float32),
    scratch_types=[
        [pltpu.VMEM((CH,), jnp.int32)] * NBUF,
        [pltpu.VMEM((CH,), jnp.int32)] * NBUF,
        [pltpu.VMEM((CH, D), jnp.float32)] * NBUF,
        [pltpu.SemaphoreType.DMA] * NBUF,
        [pltpu.SemaphoreType.DMA] * NBUF,
        [pltpu.SemaphoreType.DMA] * NBUF,
        pltpu.VMEM_SHARED((N, D), jnp.float32),
    ],
)
def _agg_kernel(hs_hbm, src_hbm, dst_hbm, out_hbm, sbuf, dbuf, rows,
                isem, gsem, ssem, agg_sh):
    c = lax.axis_index("c")
    s = lax.axis_index("s")
    wid = s * NC + c
    r0 = s * RPT
    base = wid * EPW

    def fetch_idx(i, b):
        off = base + i * CH
        pltpu.async_copy(src_hbm.at[pl.ds(off, CH)], sbuf[b], isem[b])
        pltpu.async_copy(dst_hbm.at[pl.ds(off, CH)], dbuf[b], isem[b])

    def wait_idx(b):
        pltpu.make_async_copy(src_hbm.at[pl.ds(0, CH)], sbuf[b],
                              isem[b]).wait()
        pltpu.make_async_copy(dst_hbm.at[pl.ds(0, CH)], dbuf[b],
                              isem[b]).wait()

    def start_gather(b):
        pltpu.async_copy(hs_hbm.at[sbuf[b]], rows[b], gsem[b])

    def wait_gather(b):
        pltpu.make_async_copy(hs_hbm.at[pl.ds(0, CH)], rows[b],
                              gsem[b]).wait()

    def start_scatter(b):
        pltpu.async_copy(rows[b], agg_sh.at[dbuf[b]], ssem[b], add=True)

    def wait_scatter(b):
        pltpu.make_async_copy(rows[b], agg_sh.at[dbuf[b]], ssem[b]).wait()

    for b in range(NBUF):
        fetch_idx(b, b)

    @pl.when(s < NS - 1)
    def _():
        pltpu.sync_copy(hs_hbm.at[pl.ds(r0, RPT)], agg_sh.at[pl.ds(r0, RPT)])

    @pl.when(s == NS - 1)
    def _():
        pltpu.sync_copy(hs_hbm.at[pl.ds(RLAST0, RLAST)],
                        agg_sh.at[pl.ds(RLAST0, RLAST)])

    plsc.subcore_barrier()

    def body(g, carry):
        for b in range(NBUF):
            wait_idx(b)
            start_gather(b)
        for b in range(NBUF):
            wait_gather(b)
            start_scatter(b)
        for b in range(NBUF):
            nxt = g * NBUF + b + NBUF
            wait_scatter(b)

            @pl.when(nxt < NCHUNK)
            def _():
                fetch_idx(nxt, b)

        return carry

    lax.fori_loop(0, NGROUP, body, 0)
    for b in range(NREM):
        wait_idx(b)
        start_gather(b)
    for b in range(NREM):
        wait_gather(b)
        start_scatter(b)
    for b in range(NREM):
        wait_scatter(b)
    plsc.subcore_barrier()

    @pl.when(s < NS - 1)
    def _():
        pltpu.sync_copy(agg_sh.at[pl.ds(r0, RPT)],
                        out_hbm.at[c, pl.ds(r0, RPT)])

    @pl.when(s == NS - 1)
    def _():
        pltpu.sync_copy(agg_sh.at[pl.ds(RLAST0, RLAST)],
                        out_hbm.at[c, pl.ds(RLAST0, RLAST)])


def _dinv_from(degs_block):
    deg = jnp.sum(degs_block, axis=0)[:N] + 1.0
    return lax.rsqrt(deg)


def _tc1_body(degs_ref, x_ref, w1_ref, hs_ref):
    dinv = _dinv_from(degs_ref[...])
    hs_ref[...] = jnp.dot(x_ref[...], w1_ref[...],
                          preferred_element_type=jnp.float32) * dinv[:, None]


def _tc2_body(degs_ref, agg_ref, hs1_ref, b1_ref, w2_ref, hs2_ref):
    dinv = _dinv_from(degs_ref[...])
    tot = agg_ref[0] + agg_ref[1] - hs1_ref[...]
    h1 = jnp.maximum(tot * dinv[:, None] + b1_ref[...][None, :], 0.0)
    hs2_ref[...] = jnp.dot(h1, w2_ref[...],
                           preferred_element_type=jnp.float32) * dinv[:, None]


def _tc3_body(degs_ref, agg_ref, hs2_ref, b2_ref, wh1_ref, bh1_ref, wh2_ref,
              bh2_ref, out1_ref, out2_ref, h_ref):
    dinv = _dinv_from(degs_ref[...])
    tot = agg_ref[0] + agg_ref[1] - hs2_ref[...]
    h = tot * dinv[:, None] + b2_ref[...][None, :]
    h_ref[...] = h
    out1_ref[...] = jnp.dot(h, wh1_ref[...],
                            preferred_element_type=jnp.float32) + bh1_ref[...][None, :]
    out2_ref[...] = jnp.dot(h, wh2_ref[...],
                            preferred_element_type=jnp.float32) + bh2_ref[...][None, :]


def kernel(x, edge_index, W1, b1, W2, b2, Wh1, bh1, Wh2, bh2):
    ei32 = edge_index.astype(jnp.int32)

    degs, src, dst = _deg_kernel(ei32)

    hs1 = pl.pallas_call(
        _tc1_body,
        out_shape=jax.ShapeDtypeStruct((N, D), jnp.float32),
    )(degs, x, W1)

    agg1 = _agg_kernel(hs1, src, dst)

    hs2 = pl.pallas_call(
        _tc2_body,
        out_shape=jax.ShapeDtypeStruct((N, D), jnp.float32),
    )(degs, agg1, hs1, b1, W2)

    agg2 = _agg_kernel(hs2, src, dst)

    out1, out2, h = pl.pallas_call(
        _tc3_body,
        out_shape=(
            jax.ShapeDtypeStruct((N, Wh1.shape[1]), jnp.float32),
            jax.ShapeDtypeStruct((N, Wh2.shape[1]), jnp.float32),
            jax.ShapeDtypeStruct((N, D), jnp.float32),
        ),
    )(degs, agg2, hs2, b2, Wh1, bh1, Wh2, bh2)

    return out1, out2, h
